# baseline (device time: 38535 ns/iter reference)
import jax
import jax.numpy as jnp
from jax import lax
from jax.experimental import pallas as pl
from jax.experimental.pallas import tpu as pltpu

BLOCK_M = 512


def kernel(x):
    m, n = x.shape
    nblk = m // BLOCK_M

    def body(x_ref, out_ref, acc_ref, recv_ref, send_sem, recv_sem):
        i = pl.program_id(0)

        wide = x_ref[:, 0:128]
        for s in range(1, n // 128):
            wide = wide + x_ref[:, s * 128 : (s + 1) * 128]
        acc_ref[pl.ds(i * BLOCK_M, BLOCK_M), :] = jnp.sum(
            wide, axis=1, keepdims=True
        )

        @pl.when(i == nblk - 1)
        def _():
            my_x = lax.axis_index("x")
            my_y = lax.axis_index("y")
            rdma = pltpu.make_async_remote_copy(
                src_ref=acc_ref,
                dst_ref=recv_ref,
                send_sem=send_sem,
                recv_sem=recv_sem,
                device_id=(my_x, 1 - my_y),
                device_id_type=pl.DeviceIdType.MESH,
            )
            rdma.start()
            rdma.wait()
            out_ref[...] = acc_ref[...] + recv_ref[...]

    return pl.pallas_call(
        body,
        grid=(nblk,),
        out_shape=jax.ShapeDtypeStruct((m, 1), x.dtype),
        in_specs=[
            pl.BlockSpec((BLOCK_M, n), lambda i: (i, 0), memory_space=pltpu.VMEM)
        ],
        out_specs=pl.BlockSpec((m, 1), lambda i: (0, 0), memory_space=pltpu.VMEM),
        scratch_shapes=[
            pltpu.VMEM((m, 1), x.dtype),
            pltpu.VMEM((m, 1), x.dtype),
            pltpu.SemaphoreType.DMA,
            pltpu.SemaphoreType.DMA,
        ],
    )(x)


# device time: 11778 ns/iter; 3.2718x vs baseline; 3.2718x over previous
import jax
import jax.numpy as jnp
from jax import lax
from jax.experimental import pallas as pl
from jax.experimental.pallas import tpu as pltpu

BLOCK_M = 512
LANES = 128


def kernel(x):
    m, n = x.shape
    nblk = m // BLOCK_M
    rows = BLOCK_M // LANES

    def body(x_ref, out_ref, acc_ref, recv_ref, send_sem, recv_sem):
        i = pl.program_id(0)
        my_x = lax.axis_index("x")
        my_y = lax.axis_index("y")
        peer = (my_x, 1 - my_y)

        @pl.when(i == 0)
        def _():
            barrier_sem = pltpu.get_barrier_semaphore()
            pl.semaphore_signal(
                barrier_sem, inc=1,
                device_id=peer, device_id_type=pl.DeviceIdType.MESH,
            )
            pl.semaphore_wait(barrier_sem, 1)

        wide = x_ref[:, 0:LANES]
        for s in range(1, n // LANES):
            wide = wide + x_ref[:, s * LANES : (s + 1) * LANES]
        psum = jnp.sum(wide, axis=1, keepdims=True)
        acc_ref[pl.ds(i * rows, rows), :] = psum.reshape(rows, LANES)

        @pl.when(i == nblk - 1)
        def _():
            rdma = pltpu.make_async_remote_copy(
                src_ref=acc_ref,
                dst_ref=recv_ref,
                send_sem=send_sem,
                recv_sem=recv_sem,
                device_id=peer,
                device_id_type=pl.DeviceIdType.MESH,
            )
            rdma.start()
            rdma.wait()
            out_ref[...] = acc_ref[...] + recv_ref[...]

    packed = pl.pallas_call(
        body,
        grid=(nblk,),
        out_shape=jax.ShapeDtypeStruct((m // LANES, LANES), x.dtype),
        in_specs=[
            pl.BlockSpec((BLOCK_M, n), lambda i: (i, 0), memory_space=pltpu.VMEM)
        ],
        out_specs=pl.BlockSpec(
            (m // LANES, LANES), lambda i: (0, 0), memory_space=pltpu.VMEM
        ),
        scratch_shapes=[
            pltpu.VMEM((m // LANES, LANES), x.dtype),
            pltpu.VMEM((m // LANES, LANES), x.dtype),
            pltpu.SemaphoreType.DMA,
            pltpu.SemaphoreType.DMA,
        ],
        compiler_params=pltpu.CompilerParams(collective_id=0),
    )(x)
    return jnp.reshape(packed, (m, 1))


# device time: 11548 ns/iter; 3.3369x vs baseline; 1.0199x over previous
import jax
import jax.numpy as jnp
from jax import lax
from jax.experimental import pallas as pl
from jax.experimental.pallas import tpu as pltpu

BLOCK_M = 512
LANES = 128


def kernel(x):
    m, n = x.shape
    nblk = m // BLOCK_M
    rows = BLOCK_M // LANES

    def body(x_ref, out_ref, acc_ref, recv_ref, send_sems, recv_sems):
        i = pl.program_id(0)
        my_x = lax.axis_index("x")
        my_y = lax.axis_index("y")
        peer = (my_x, 1 - my_y)

        @pl.when(i == 0)
        def _():
            barrier_sem = pltpu.get_barrier_semaphore()
            pl.semaphore_signal(
                barrier_sem, inc=1,
                device_id=peer, device_id_type=pl.DeviceIdType.MESH,
            )
            pl.semaphore_wait(barrier_sem, 1)

        wide = x_ref[:, 0:LANES]
        for s in range(1, n // LANES):
            wide = wide + x_ref[:, s * LANES : (s + 1) * LANES]
        psum = jnp.sum(wide, axis=1, keepdims=True)
        acc_ref[pl.ds(i * rows, rows), :] = psum.reshape(rows, LANES)

        half_blk = nblk // 2
        half_rows = half_blk * rows

        def half_rdma(h):
            return pltpu.make_async_remote_copy(
                src_ref=acc_ref.at[pl.ds(h * half_rows, half_rows), :],
                dst_ref=recv_ref.at[pl.ds(h * half_rows, half_rows), :],
                send_sem=send_sems.at[h],
                recv_sem=recv_sems.at[h],
                device_id=peer,
                device_id_type=pl.DeviceIdType.MESH,
            )

        @pl.when(i == half_blk - 1)
        def _():
            half_rdma(0).start()

        @pl.when(i == nblk - 1)
        def _():
            half_rdma(1).start()
            half_rdma(1).wait()
            half_rdma(0).wait()
            out_ref[...] = acc_ref[...] + recv_ref[...]

    packed = pl.pallas_call(
        body,
        grid=(nblk,),
        out_shape=jax.ShapeDtypeStruct((m // LANES, LANES), x.dtype),
        in_specs=[
            pl.BlockSpec((BLOCK_M, n), lambda i: (i, 0), memory_space=pltpu.VMEM)
        ],
        out_specs=pl.BlockSpec(
            (m // LANES, LANES), lambda i: (0, 0), memory_space=pltpu.VMEM
        ),
        scratch_shapes=[
            pltpu.VMEM((m // LANES, LANES), x.dtype),
            pltpu.VMEM((m // LANES, LANES), x.dtype),
            pltpu.SemaphoreType.DMA((2,)),
            pltpu.SemaphoreType.DMA((2,)),
        ],
        compiler_params=pltpu.CompilerParams(collective_id=0),
    )(x)
    return jnp.reshape(packed, (m, 1))
